# baseline (speedup 1.0000x reference)
import functools

import jax
import jax.numpy as jnp
from jax import lax
from jax.experimental import pallas as pl
from jax.experimental.pallas import tpu as pltpu
from jax.experimental.pallas import tpu_sc as plsc

NUM_CONCEPTS = 1000000
DIM = 32
BATCH = 16384
EPS = 1e-23

_NC = 2
_NS = 16
_NW = _NC * _NS
_BPW = BATCH // _NW
_GCH = 128
_NG = _BPW // _GCH
_SROW = 4 * DIM


_H = 1 << 19
_TBLK = 2048
_TGRID = _H // _TBLK
_TLAST = NUM_CONCEPTS // _TBLK


def _tc_pack_body(xa_ref, xb_ref, out_ref):
    out_ref[...] = jnp.concatenate(
        [xa_ref[...].T, xb_ref[...].T], axis=1)


def _tc_pack(embT):
    return pl.pallas_call(
        _tc_pack_body,
        grid=(_TGRID,),
        in_specs=[
            pl.BlockSpec((2 * DIM, _TBLK), lambda g: (0, g)),
            pl.BlockSpec((2 * DIM, _TBLK),
                         lambda g: (0, jnp.minimum(g + _TGRID, _TLAST))),
        ],
        out_specs=pl.BlockSpec((_TBLK, 4 * DIM), lambda g: (g, 0)),
        out_shape=jax.ShapeDtypeStruct((_H, 4 * DIM), jnp.float32),
    )(embT, embT)


def _sc_gather(sidx_i, sidx_j, emb2):
    mesh = plsc.VectorSubcoreMesh(core_axis_name="c", subcore_axis_name="s")

    @functools.partial(
        pl.kernel,
        mesh=mesh,
        out_type=jax.ShapeDtypeStruct((BATCH, 2 * _SROW), jnp.float32),
        scratch_types=[
            pltpu.VMEM((_BPW,), jnp.int32),
            pltpu.VMEM((_BPW,), jnp.int32),
            pltpu.VMEM((_BPW, _SROW), jnp.float32),
            pltpu.SemaphoreType.DMA,
        ],
    )
    def gather_kernel(sidx_i_hbm, sidx_j_hbm, emb2_hbm, out_hbm,
                      ii_v, ij_v, rows_v, sem):
        wid = lax.axis_index("s") * _NC + lax.axis_index("c")
        base = wid * _BPW
        pltpu.sync_copy(sidx_i_hbm.at[pl.ds(base, _BPW)], ii_v)
        pltpu.sync_copy(sidx_j_hbm.at[pl.ds(base, _BPW)], ij_v)
        for idx_v, off in ((ii_v, 0), (ij_v, _SROW)):
            copies = []
            for g in range(_NG):
                sl = pl.ds(g * _GCH, _GCH)
                copies.append(pltpu.async_copy(
                    emb2_hbm.at[idx_v.at[sl]], rows_v.at[sl], sem))
            for c in copies:
                c.wait()
            pltpu.sync_copy(rows_v, out_hbm.at[pl.ds(base, _BPW), pl.ds(off, _SROW)])

    return gather_kernel(sidx_i, sidx_j, emb2)


def _softplus(x):
    return jnp.maximum(x, 0.0) + jnp.log1p(jnp.exp(-jnp.abs(x)))


def _half(sup, par):
    lo = sup[:, 0:2 * DIM]
    hi = sup[:, 2 * DIM:4 * DIM]
    return jnp.where(par, hi, lo)


def _tc_math_body(pi_ref, pj_ref, x_ref, out_ref):
    x = x_ref[...]
    par_i = pi_ref[...] != 0
    par_j = pj_ref[...] != 0
    ti = _half(x[:, 0:_SROW], par_i)
    tj = _half(x[:, _SROW:2 * _SROW], par_j)
    z_i = ti[:, 0:DIM]
    d_i = ti[:, DIM:2 * DIM]
    z_j = tj[:, 0:DIM]
    d_j = tj[:, DIM:2 * DIM]
    sp_di = _softplus(d_i)
    sp_dj = _softplus(d_j)
    Z_i = z_i + sp_di
    Z_j = z_j + sp_dj
    side_int = _softplus(jnp.minimum(Z_i, Z_j) - jnp.maximum(z_i, z_j))
    side_j = _softplus(sp_dj)
    ratio = (side_int + EPS) / (side_j + EPS)
    r = ratio
    w = DIM
    while w > 1:
        w //= 2
        r = r[:, :w] * r[:, w:]
    p = r[:, 0]
    out_ref[...] = jnp.clip(p, 1e-7, 1.0 - 1e-7)


def _tc_math(par_i, par_j, rows):
    blk = 2048
    grid = BATCH // blk
    return pl.pallas_call(
        _tc_math_body,
        grid=(grid,),
        in_specs=[
            pl.BlockSpec((blk, 1), lambda b: (b, 0)),
            pl.BlockSpec((blk, 1), lambda b: (b, 0)),
            pl.BlockSpec((blk, 2 * _SROW), lambda b: (b, 0)),
        ],
        out_specs=pl.BlockSpec((blk,), lambda b: (b,)),
        out_shape=jax.ShapeDtypeStruct((BATCH,), jnp.float32),
    )(par_i, par_j, rows)


def kernel(idx_i, idx_j, emb):
    idx_i = idx_i.astype(jnp.int32)
    idx_j = idx_j.astype(jnp.int32)
    emb2 = _tc_pack(emb.T)
    si = idx_i & (_H - 1)
    sj = idx_j & (_H - 1)
    pi = (idx_i >> 19).reshape(BATCH, 1)
    pj = (idx_j >> 19).reshape(BATCH, 1)
    rows = _sc_gather(si, sj, emb2)
    return _tc_math(pi, pj, rows)


# speedup vs baseline: 1.2090x; 1.2090x over previous
import functools

import jax
import jax.numpy as jnp
from jax import lax
from jax.experimental import pallas as pl
from jax.experimental.pallas import tpu as pltpu
from jax.experimental.pallas import tpu_sc as plsc

NUM_CONCEPTS = 1000000
DIM = 32
BATCH = 16384
EPS = 1e-23

_NC = 2
_NS = 16
_NW = _NC * _NS
_BPW = BATCH // _NW
_GCH = 128
_NG = _BPW // _GCH
_SROW = 4 * DIM


_H = 1 << 19
_TBLK = 4096
_TGRID = _H // _TBLK
_TLAST = NUM_CONCEPTS // _TBLK


def _tc_pack_body(xa_ref, xb_ref, out_ref):
    out_ref[...] = jnp.concatenate(
        [xa_ref[...].T, xb_ref[...].T], axis=1)


def _tc_pack(embT):
    return pl.pallas_call(
        _tc_pack_body,
        grid=(_TGRID,),
        in_specs=[
            pl.BlockSpec((2 * DIM, _TBLK), lambda g: (0, g)),
            pl.BlockSpec((2 * DIM, _TBLK),
                         lambda g: (0, jnp.minimum(g + _TGRID, _TLAST))),
        ],
        out_specs=pl.BlockSpec((_TBLK, 4 * DIM), lambda g: (g, 0)),
        out_shape=jax.ShapeDtypeStruct((_H, 4 * DIM), jnp.float32),
    )(embT, embT)


def _sc_gather(sidx_i, sidx_j, emb2):
    mesh = plsc.VectorSubcoreMesh(core_axis_name="c", subcore_axis_name="s")

    @functools.partial(
        pl.kernel,
        mesh=mesh,
        out_type=jax.ShapeDtypeStruct((BATCH, 2 * _SROW), jnp.float32),
        scratch_types=[
            pltpu.VMEM((_BPW,), jnp.int32),
            pltpu.VMEM((_BPW,), jnp.int32),
            pltpu.VMEM((_BPW, _SROW), jnp.float32),
            pltpu.SemaphoreType.DMA,
        ],
    )
    def gather_kernel(sidx_i_hbm, sidx_j_hbm, emb2_hbm, out_hbm,
                      ii_v, ij_v, rows_v, sem):
        wid = lax.axis_index("s") * _NC + lax.axis_index("c")
        base = wid * _BPW
        pltpu.sync_copy(sidx_i_hbm.at[pl.ds(base, _BPW)], ii_v)
        pltpu.sync_copy(sidx_j_hbm.at[pl.ds(base, _BPW)], ij_v)
        for idx_v, off in ((ii_v, 0), (ij_v, _SROW)):
            copies = []
            for g in range(_NG):
                sl = pl.ds(g * _GCH, _GCH)
                copies.append(pltpu.async_copy(
                    emb2_hbm.at[idx_v.at[sl]], rows_v.at[sl], sem))
            for c in copies:
                c.wait()
            pltpu.sync_copy(rows_v, out_hbm.at[pl.ds(base, _BPW), pl.ds(off, _SROW)])

    return gather_kernel(sidx_i, sidx_j, emb2)


def _softplus(x):
    return jnp.maximum(x, 0.0) + jnp.log1p(jnp.exp(-jnp.abs(x)))


def _half(sup, par):
    lo = sup[:, 0:2 * DIM]
    hi = sup[:, 2 * DIM:4 * DIM]
    return jnp.where(par, hi, lo)


def _tc_math_body(pi_ref, pj_ref, x_ref, out_ref):
    x = x_ref[...]
    par_i = pi_ref[...] != 0
    par_j = pj_ref[...] != 0
    ti = _half(x[:, 0:_SROW], par_i)
    tj = _half(x[:, _SROW:2 * _SROW], par_j)
    z_i = ti[:, 0:DIM]
    d_i = ti[:, DIM:2 * DIM]
    z_j = tj[:, 0:DIM]
    d_j = tj[:, DIM:2 * DIM]
    sp_di = _softplus(d_i)
    sp_dj = _softplus(d_j)
    Z_i = z_i + sp_di
    Z_j = z_j + sp_dj
    side_int = _softplus(jnp.minimum(Z_i, Z_j) - jnp.maximum(z_i, z_j))
    side_j = _softplus(sp_dj)
    ratio = (side_int + EPS) / (side_j + EPS)
    r = ratio
    w = DIM
    while w > 1:
        w //= 2
        r = r[:, :w] * r[:, w:]
    p = r[:, 0]
    out_ref[...] = jnp.clip(p, 1e-7, 1.0 - 1e-7)


def _tc_math(par_i, par_j, rows):
    blk = 2048
    grid = BATCH // blk
    return pl.pallas_call(
        _tc_math_body,
        grid=(grid,),
        in_specs=[
            pl.BlockSpec((blk, 1), lambda b: (b, 0)),
            pl.BlockSpec((blk, 1), lambda b: (b, 0)),
            pl.BlockSpec((blk, 2 * _SROW), lambda b: (b, 0)),
        ],
        out_specs=pl.BlockSpec((blk,), lambda b: (b,)),
        out_shape=jax.ShapeDtypeStruct((BATCH,), jnp.float32),
    )(par_i, par_j, rows)


def kernel(idx_i, idx_j, emb):
    idx_i = idx_i.astype(jnp.int32)
    idx_j = idx_j.astype(jnp.int32)
    emb2 = _tc_pack(emb.T)
    si = idx_i & (_H - 1)
    sj = idx_j & (_H - 1)
    pi = (idx_i >> 19).reshape(BATCH, 1)
    pj = (idx_j >> 19).reshape(BATCH, 1)
    rows = _sc_gather(si, sj, emb2)
    return _tc_math(pi, pj, rows)


# speedup vs baseline: 1.3387x; 1.1072x over previous
import functools

import jax
import jax.numpy as jnp
from jax import lax
from jax.experimental import pallas as pl
from jax.experimental.pallas import tpu as pltpu
from jax.experimental.pallas import tpu_sc as plsc

NUM_CONCEPTS = 1000000
DIM = 32
BATCH = 16384
EPS = 1e-23

_NC = 2
_NS = 16
_NW = _NC * _NS
_BPW = BATCH // _NW
_GCH = 128
_NG = _BPW // _GCH
_SROW = 4 * DIM


_H = 1 << 19
_TBLK = 8192
_TGRID = _H // _TBLK
_TLAST = NUM_CONCEPTS // _TBLK


def _tc_pack_body(xa_ref, xb_ref, out_ref):
    out_ref[...] = jnp.concatenate(
        [xa_ref[...].T, xb_ref[...].T], axis=1)


def _tc_pack(embT):
    return pl.pallas_call(
        _tc_pack_body,
        grid=(_TGRID,),
        in_specs=[
            pl.BlockSpec((2 * DIM, _TBLK), lambda g: (0, g)),
            pl.BlockSpec((2 * DIM, _TBLK),
                         lambda g: (0, jnp.minimum(g + _TGRID, _TLAST))),
        ],
        out_specs=pl.BlockSpec((_TBLK, 4 * DIM), lambda g: (g, 0)),
        out_shape=jax.ShapeDtypeStruct((_H, 4 * DIM), jnp.float32),
    )(embT, embT)


def _sc_gather(sidx_i, sidx_j, emb2):
    mesh = plsc.VectorSubcoreMesh(core_axis_name="c", subcore_axis_name="s")

    @functools.partial(
        pl.kernel,
        mesh=mesh,
        out_type=jax.ShapeDtypeStruct((BATCH, 2 * _SROW), jnp.float32),
        scratch_types=[
            pltpu.VMEM((_BPW,), jnp.int32),
            pltpu.VMEM((_BPW,), jnp.int32),
            pltpu.VMEM((_BPW, _SROW), jnp.float32),
            pltpu.SemaphoreType.DMA,
        ],
    )
    def gather_kernel(sidx_i_hbm, sidx_j_hbm, emb2_hbm, out_hbm,
                      ii_v, ij_v, rows_v, sem):
        wid = lax.axis_index("s") * _NC + lax.axis_index("c")
        base = wid * _BPW
        pltpu.sync_copy(sidx_i_hbm.at[pl.ds(base, _BPW)], ii_v)
        pltpu.sync_copy(sidx_j_hbm.at[pl.ds(base, _BPW)], ij_v)
        for idx_v, off in ((ii_v, 0), (ij_v, _SROW)):
            copies = []
            for g in range(_NG):
                sl = pl.ds(g * _GCH, _GCH)
                copies.append(pltpu.async_copy(
                    emb2_hbm.at[idx_v.at[sl]], rows_v.at[sl], sem))
            for c in copies:
                c.wait()
            pltpu.sync_copy(rows_v, out_hbm.at[pl.ds(base, _BPW), pl.ds(off, _SROW)])

    return gather_kernel(sidx_i, sidx_j, emb2)


def _softplus(x):
    return jnp.maximum(x, 0.0) + jnp.log1p(jnp.exp(-jnp.abs(x)))


def _half(sup, par):
    lo = sup[:, 0:2 * DIM]
    hi = sup[:, 2 * DIM:4 * DIM]
    return jnp.where(par, hi, lo)


def _tc_math_body(pi_ref, pj_ref, x_ref, out_ref):
    x = x_ref[...]
    par_i = pi_ref[...] != 0
    par_j = pj_ref[...] != 0
    ti = _half(x[:, 0:_SROW], par_i)
    tj = _half(x[:, _SROW:2 * _SROW], par_j)
    z_i = ti[:, 0:DIM]
    d_i = ti[:, DIM:2 * DIM]
    z_j = tj[:, 0:DIM]
    d_j = tj[:, DIM:2 * DIM]
    sp_di = _softplus(d_i)
    sp_dj = _softplus(d_j)
    Z_i = z_i + sp_di
    Z_j = z_j + sp_dj
    side_int = _softplus(jnp.minimum(Z_i, Z_j) - jnp.maximum(z_i, z_j))
    side_j = _softplus(sp_dj)
    ratio = (side_int + EPS) / (side_j + EPS)
    r = ratio
    w = DIM
    while w > 1:
        w //= 2
        r = r[:, :w] * r[:, w:]
    p = r[:, 0]
    out_ref[...] = jnp.clip(p, 1e-7, 1.0 - 1e-7)


def _tc_math(par_i, par_j, rows):
    blk = 2048
    grid = BATCH // blk
    return pl.pallas_call(
        _tc_math_body,
        grid=(grid,),
        in_specs=[
            pl.BlockSpec((blk, 1), lambda b: (b, 0)),
            pl.BlockSpec((blk, 1), lambda b: (b, 0)),
            pl.BlockSpec((blk, 2 * _SROW), lambda b: (b, 0)),
        ],
        out_specs=pl.BlockSpec((blk,), lambda b: (b,)),
        out_shape=jax.ShapeDtypeStruct((BATCH,), jnp.float32),
    )(par_i, par_j, rows)


def kernel(idx_i, idx_j, emb):
    idx_i = idx_i.astype(jnp.int32)
    idx_j = idx_j.astype(jnp.int32)
    emb2 = _tc_pack(emb.T)
    si = idx_i & (_H - 1)
    sj = idx_j & (_H - 1)
    pi = (idx_i >> 19).reshape(BATCH, 1)
    pj = (idx_j >> 19).reshape(BATCH, 1)
    rows = _sc_gather(si, sj, emb2)
    return _tc_math(pi, pj, rows)


# speedup vs baseline: 1.3998x; 1.0457x over previous
import functools

import jax
import jax.numpy as jnp
from jax import lax
from jax.experimental import pallas as pl
from jax.experimental.pallas import tpu as pltpu
from jax.experimental.pallas import tpu_sc as plsc

NUM_CONCEPTS = 1000000
DIM = 32
BATCH = 16384
EPS = 1e-23

_NC = 2
_NS = 16
_NW = _NC * _NS
_BPW = BATCH // _NW
_GCH = 128
_NG = _BPW // _GCH
_SROW = 4 * DIM


_H = 1 << 19
_TBLK = 16384
_TGRID = _H // _TBLK
_TLAST = NUM_CONCEPTS // _TBLK


def _tc_pack_body(xa_ref, xb_ref, out_ref):
    out_ref[...] = jnp.concatenate(
        [xa_ref[...].T, xb_ref[...].T], axis=1)


def _tc_pack(embT):
    return pl.pallas_call(
        _tc_pack_body,
        grid=(_TGRID,),
        in_specs=[
            pl.BlockSpec((2 * DIM, _TBLK), lambda g: (0, g)),
            pl.BlockSpec((2 * DIM, _TBLK),
                         lambda g: (0, jnp.minimum(g + _TGRID, _TLAST))),
        ],
        out_specs=pl.BlockSpec((_TBLK, 4 * DIM), lambda g: (g, 0)),
        out_shape=jax.ShapeDtypeStruct((_H, 4 * DIM), jnp.float32),
    )(embT, embT)


def _sc_gather(sidx_i, sidx_j, emb2):
    mesh = plsc.VectorSubcoreMesh(core_axis_name="c", subcore_axis_name="s")

    @functools.partial(
        pl.kernel,
        mesh=mesh,
        out_type=jax.ShapeDtypeStruct((BATCH, 2 * _SROW), jnp.float32),
        scratch_types=[
            pltpu.VMEM((_BPW,), jnp.int32),
            pltpu.VMEM((_BPW,), jnp.int32),
            pltpu.VMEM((_BPW, _SROW), jnp.float32),
            pltpu.SemaphoreType.DMA,
        ],
    )
    def gather_kernel(sidx_i_hbm, sidx_j_hbm, emb2_hbm, out_hbm,
                      ii_v, ij_v, rows_v, sem):
        wid = lax.axis_index("s") * _NC + lax.axis_index("c")
        base = wid * _BPW
        pltpu.sync_copy(sidx_i_hbm.at[pl.ds(base, _BPW)], ii_v)
        pltpu.sync_copy(sidx_j_hbm.at[pl.ds(base, _BPW)], ij_v)
        for idx_v, off in ((ii_v, 0), (ij_v, _SROW)):
            copies = []
            for g in range(_NG):
                sl = pl.ds(g * _GCH, _GCH)
                copies.append(pltpu.async_copy(
                    emb2_hbm.at[idx_v.at[sl]], rows_v.at[sl], sem))
            for c in copies:
                c.wait()
            pltpu.sync_copy(rows_v, out_hbm.at[pl.ds(base, _BPW), pl.ds(off, _SROW)])

    return gather_kernel(sidx_i, sidx_j, emb2)


def _softplus(x):
    return jnp.maximum(x, 0.0) + jnp.log1p(jnp.exp(-jnp.abs(x)))


def _half(sup, par):
    lo = sup[:, 0:2 * DIM]
    hi = sup[:, 2 * DIM:4 * DIM]
    return jnp.where(par, hi, lo)


def _tc_math_body(pi_ref, pj_ref, x_ref, out_ref):
    x = x_ref[...]
    par_i = pi_ref[...] != 0
    par_j = pj_ref[...] != 0
    ti = _half(x[:, 0:_SROW], par_i)
    tj = _half(x[:, _SROW:2 * _SROW], par_j)
    z_i = ti[:, 0:DIM]
    d_i = ti[:, DIM:2 * DIM]
    z_j = tj[:, 0:DIM]
    d_j = tj[:, DIM:2 * DIM]
    sp_di = _softplus(d_i)
    sp_dj = _softplus(d_j)
    Z_i = z_i + sp_di
    Z_j = z_j + sp_dj
    side_int = _softplus(jnp.minimum(Z_i, Z_j) - jnp.maximum(z_i, z_j))
    side_j = _softplus(sp_dj)
    ratio = (side_int + EPS) / (side_j + EPS)
    r = ratio
    w = DIM
    while w > 1:
        w //= 2
        r = r[:, :w] * r[:, w:]
    p = r[:, 0]
    out_ref[...] = jnp.clip(p, 1e-7, 1.0 - 1e-7)


def _tc_math(par_i, par_j, rows):
    blk = 2048
    grid = BATCH // blk
    return pl.pallas_call(
        _tc_math_body,
        grid=(grid,),
        in_specs=[
            pl.BlockSpec((blk, 1), lambda b: (b, 0)),
            pl.BlockSpec((blk, 1), lambda b: (b, 0)),
            pl.BlockSpec((blk, 2 * _SROW), lambda b: (b, 0)),
        ],
        out_specs=pl.BlockSpec((blk,), lambda b: (b,)),
        out_shape=jax.ShapeDtypeStruct((BATCH,), jnp.float32),
    )(par_i, par_j, rows)


def kernel(idx_i, idx_j, emb):
    idx_i = idx_i.astype(jnp.int32)
    idx_j = idx_j.astype(jnp.int32)
    emb2 = _tc_pack(emb.T)
    si = idx_i & (_H - 1)
    sj = idx_j & (_H - 1)
    pi = (idx_i >> 19).reshape(BATCH, 1)
    pj = (idx_j >> 19).reshape(BATCH, 1)
    rows = _sc_gather(si, sj, emb2)
    return _tc_math(pi, pj, rows)
